# Initial kernel scaffold; baseline (speedup 1.0000x reference)
import functools

import jax
import jax.numpy as jnp
from jax import lax
from jax.experimental import pallas as pl
from jax.experimental.pallas import tpu as pltpu
from jax.experimental.pallas import tpu_sc as plsc

NC = 2
NS = 16
LANES = 16


def _prep0_body(x_ref, w_ref, o_ref):
    o_ref[...] = jnp.dot(x_ref[...], w_ref[...],
                         preferred_element_type=jnp.float32)


def _tc_node_matmul(x, w, bn):
    n, d = x.shape
    return pl.pallas_call(
        _prep0_body,
        grid=(n // bn,),
        in_specs=[
            pl.BlockSpec((bn, d), lambda i: (i, 0)),
            pl.BlockSpec((d, w.shape[1]), lambda i: (0, 0)),
        ],
        out_specs=pl.BlockSpec((bn, w.shape[1]), lambda i: (i, 0)),
        out_shape=jax.ShapeDtypeStruct((n, w.shape[1]), jnp.float32),
    )(x, w)


def _edge_prep2_body(ea_ref, wme_ref, bm_ref, wee_ref, be_ref,
                     eaw_ref, eaw2_ref):
    ea = ea_ref[...]
    eaw_ref[...] = jnp.dot(ea, wme_ref[...],
                           preferred_element_type=jnp.float32) + bm_ref[...]
    eaw2_ref[...] = jnp.dot(ea, wee_ref[...],
                            preferred_element_type=jnp.float32) + be_ref[...]


def _edge_prep1_body(ea_ref, wme_ref, bm_ref, eaw_ref):
    eaw_ref[...] = jnp.dot(ea_ref[...], wme_ref[...],
                           preferred_element_type=jnp.float32) + bm_ref[...]


def _edge_prep(ea, wme, bmv, wee, bev, be_blk):
    e, de = ea.shape
    d = wme.shape[1]
    grid = (e // be_blk,)
    if wee is None:
        return pl.pallas_call(
            _edge_prep1_body,
            grid=grid,
            in_specs=[
                pl.BlockSpec((be_blk, de), lambda i: (i, 0)),
                pl.BlockSpec((de, d), lambda i: (0, 0)),
                pl.BlockSpec((1, d), lambda i: (0, 0)),
            ],
            out_specs=pl.BlockSpec((be_blk, d), lambda i: (i, 0)),
            out_shape=jax.ShapeDtypeStruct((e, d), jnp.float32),
        )(ea, wme, bmv)
    return pl.pallas_call(
        _edge_prep2_body,
        grid=grid,
        in_specs=[
            pl.BlockSpec((be_blk, de), lambda i: (i, 0)),
            pl.BlockSpec((de, d), lambda i: (0, 0)),
            pl.BlockSpec((1, d), lambda i: (0, 0)),
            pl.BlockSpec((de, de), lambda i: (0, 0)),
            pl.BlockSpec((1, de), lambda i: (0, 0)),
        ],
        out_specs=[
            pl.BlockSpec((be_blk, d), lambda i: (i, 0)),
            pl.BlockSpec((be_blk, de), lambda i: (i, 0)),
        ],
        out_shape=[
            jax.ShapeDtypeStruct((e, d), jnp.float32),
            jax.ShapeDtypeStruct((e, de), jnp.float32),
        ],
    )(ea, wme, bmv, wee, bev)


def _update2_body(sp_ref, cp_ref, h_ref, waa_ref, wah_ref, ba_ref,
                  wmxn_ref, wei_ref, wej_ref,
                  hn_ref, hxn_ref, hi_ref, hj_ref):
    s = sp_ref[0] + sp_ref[1]
    cnt = cp_ref[0, :, 0:1] + cp_ref[1, :, 0:1]
    agg = s * (1.0 / jnp.maximum(cnt, 1.0))
    u = jnp.dot(agg, waa_ref[...], preferred_element_type=jnp.float32)
    u = u + jnp.dot(h_ref[...], wah_ref[...],
                    preferred_element_type=jnp.float32)
    u = jnp.maximum(u + ba_ref[...], 0.0)
    nn = jnp.sqrt(jnp.sum(u * u, axis=1, keepdims=True))
    hv = u / jnp.maximum(nn, 1e-12)
    hn_ref[...] = hv
    hxn_ref[...] = jnp.dot(hv, wmxn_ref[...],
                           preferred_element_type=jnp.float32)
    hi_ref[...] = jnp.dot(hv, wei_ref[...],
                          preferred_element_type=jnp.float32)
    hj_ref[...] = jnp.dot(hv, wej_ref[...],
                          preferred_element_type=jnp.float32)


def _update1_body(sp_ref, cp_ref, h_ref, waa_ref, wah_ref, ba_ref, hn_ref):
    s = sp_ref[0] + sp_ref[1]
    cnt = cp_ref[0, :, 0:1] + cp_ref[1, :, 0:1]
    agg = s * (1.0 / jnp.maximum(cnt, 1.0))
    u = jnp.dot(agg, waa_ref[...], preferred_element_type=jnp.float32)
    u = u + jnp.dot(h_ref[...], wah_ref[...],
                    preferred_element_type=jnp.float32)
    u = jnp.maximum(u + ba_ref[...], 0.0)
    nn = jnp.sqrt(jnp.sum(u * u, axis=1, keepdims=True))
    hn_ref[...] = u / jnp.maximum(nn, 1e-12)


def _update(sp, cp, h, waa, wah, bav, wmxn, wei, wej, bn):
    n, d = h.shape
    de = cp.shape[2]
    grid = (n // bn,)
    common_in = [
        pl.BlockSpec((NC, bn, d), lambda i: (0, i, 0)),
        pl.BlockSpec((NC, bn, de), lambda i: (0, i, 0)),
        pl.BlockSpec((bn, d), lambda i: (i, 0)),
        pl.BlockSpec((d, d), lambda i: (0, 0)),
        pl.BlockSpec((d, d), lambda i: (0, 0)),
        pl.BlockSpec((1, d), lambda i: (0, 0)),
    ]
    if wmxn is None:
        return pl.pallas_call(
            _update1_body,
            grid=grid,
            in_specs=common_in,
            out_specs=pl.BlockSpec((bn, d), lambda i: (i, 0)),
            out_shape=jax.ShapeDtypeStruct((n, d), jnp.float32),
        )(sp, cp, h, waa, wah, bav)
    return pl.pallas_call(
        _update2_body,
        grid=grid,
        in_specs=common_in + [
            pl.BlockSpec((d, d), lambda i: (0, 0)),
            pl.BlockSpec((d, de), lambda i: (0, 0)),
            pl.BlockSpec((d, de), lambda i: (0, 0)),
        ],
        out_specs=[
            pl.BlockSpec((bn, d), lambda i: (i, 0)),
            pl.BlockSpec((bn, d), lambda i: (i, 0)),
            pl.BlockSpec((bn, de), lambda i: (i, 0)),
            pl.BlockSpec((bn, de), lambda i: (i, 0)),
        ],
        out_shape=[
            jax.ShapeDtypeStruct((n, d), jnp.float32),
            jax.ShapeDtypeStruct((n, d), jnp.float32),
            jax.ShapeDtypeStruct((n, de), jnp.float32),
            jax.ShapeDtypeStruct((n, de), jnp.float32),
        ],
    )(sp, cp, h, waa, wah, bav, wmxn, wei, wej)


def _make_msg_kernel(nn, dd, ee, with_cnt):
    w = NC * NS
    ept = ee // w
    ch = 80
    nchunk = ept // ch
    rpt = nn // NS
    zr = 125
    nz = rpt // zr
    assert ept % ch == 0 and rpt % zr == 0 and dd % LANES == 0

    mesh = plsc.VectorSubcoreMesh(core_axis_name="c", subcore_axis_name="s")

    out_type = [jax.ShapeDtypeStruct((NC, nn, dd), jnp.float32)]
    scratch = [
        pltpu.VMEM((ch,), jnp.int32),
        pltpu.VMEM((1, ch), jnp.int32),
        pltpu.VMEM((ch, dd), jnp.float32),
        pltpu.VMEM((ch, dd), jnp.float32),
        pltpu.VMEM((zr, dd), jnp.float32),
        pltpu.VMEM_SHARED((nn, dd), jnp.float32),
        pltpu.SemaphoreType.DMA,
    ]
    if with_cnt:
        out_type.append(jax.ShapeDtypeStruct((NC, nn, LANES), jnp.float32))
        scratch += [
            pltpu.VMEM((ch, LANES), jnp.float32),
            pltpu.VMEM((zr, LANES), jnp.float32),
            pltpu.VMEM_SHARED((nn, LANES), jnp.float32),
        ]

    def body(hx, eaw, srcr, dstr, out_s, *rest):
        if with_cnt:
            (out_c, srci, dsti, rows, eawb, zbuf, acc, sem,
             ones, zcnt, acccnt) = rest
        else:
            srci, dsti, rows, eawb, zbuf, acc, sem = rest
        c = lax.axis_index("c")
        s = lax.axis_index("s")
        ncol = dd // LANES

        def zrow(r, carry):
            for cc in range(ncol):
                zbuf[r, cc * LANES:(cc + 1) * LANES] = jnp.zeros(
                    (LANES,), jnp.float32)
            return carry
        lax.fori_loop(0, zr, zrow, 0)

        base_row = s * rpt
        for z in range(nz):
            pltpu.sync_copy(zbuf, acc.at[pl.ds(base_row + z * zr, zr)])

        if with_cnt:
            def fill(r, carry):
                ones[r, 0:LANES] = jnp.ones((LANES,), jnp.float32)
                return carry
            lax.fori_loop(0, ch, fill, 0)

            def zcrow(r, carry):
                zcnt[r, 0:LANES] = jnp.zeros((LANES,), jnp.float32)
                return carry
            lax.fori_loop(0, zr, zcrow, 0)
            for z in range(nz):
                pltpu.sync_copy(zcnt, acccnt.at[pl.ds(base_row + z * zr, zr)])

        plsc.subcore_barrier()

        ebase = (c * NS + s) * ept

        def chunk(j, carry):
            b = ebase + j * ch
            pltpu.sync_copy(srcr.at[pl.ds(b, ch)], srci)
            pltpu.sync_copy(dstr.at[pl.ds(b, ch)], dsti.at[0])
            pltpu.async_copy(hx.at[srci], rows, sem).wait()
            pltpu.sync_copy(eaw.at[pl.ds(b, ch)], eawb)

            def crow(r, carry2):
                for cc in range(ncol):
                    sl = slice(cc * LANES, (cc + 1) * LANES)
                    rows[r, sl] = jnp.maximum(rows[r, sl] + eawb[r, sl], 0.0)
                return carry2
            lax.fori_loop(0, ch, crow, 0)

            pltpu.sync_copy(rows, acc.at[dsti.at[0]], add=True)
            if with_cnt:
                pltpu.sync_copy(ones, acccnt.at[dsti.at[0]], add=True)
            return carry
        lax.fori_loop(0, nchunk, chunk, 0)

        plsc.subcore_barrier()
        pltpu.sync_copy(acc.at[pl.ds(base_row, rpt)],
                        out_s.at[c, pl.ds(base_row, rpt)])
        if with_cnt:
            pltpu.sync_copy(acccnt.at[pl.ds(base_row, rpt)],
                            out_c.at[c, pl.ds(base_row, rpt)])

    return pl.kernel(body, out_type=out_type, mesh=mesh,
                     scratch_types=scratch)


def _make_edge_update_kernel(nn, de, ee):
    w = NC * NS
    ept = ee // w
    ch = 80
    nchunk = ept // ch
    assert ept % ch == 0 and de == LANES

    mesh = plsc.VectorSubcoreMesh(core_axis_name="c", subcore_axis_name="s")
    out_type = jax.ShapeDtypeStruct((ee, de), jnp.float32)
    scratch = [
        pltpu.VMEM((ch,), jnp.int32),
        pltpu.VMEM((ch,), jnp.int32),
        pltpu.VMEM((ch, de), jnp.float32),
        pltpu.VMEM((ch, de), jnp.float32),
        pltpu.VMEM((ch, de), jnp.float32),
        pltpu.SemaphoreType.DMA,
        pltpu.SemaphoreType.DMA,
    ]

    def body(hi, hj, eaw2, srcr, dstr, out, srci, dsti, g1, g2, eb,
             sem1, sem2):
        c = lax.axis_index("c")
        s = lax.axis_index("s")
        ebase = (c * NS + s) * ept

        def chunk(j, carry):
            b = ebase + j * ch
            pltpu.sync_copy(srcr.at[pl.ds(b, ch)], srci)
            pltpu.sync_copy(dstr.at[pl.ds(b, ch)], dsti)
            cp1 = pltpu.async_copy(hi.at[srci], g1, sem1)
            cp2 = pltpu.async_copy(hj.at[dsti], g2, sem2)
            pltpu.sync_copy(eaw2.at[pl.ds(b, ch)], eb)
            cp1.wait()
            cp2.wait()

            def crow(r, carry2):
                v = eb[r, 0:LANES] + g1[r, 0:LANES] + g2[r, 0:LANES]
                eb[r, 0:LANES] = jnp.maximum(v, 0.0)
                return carry2
            lax.fori_loop(0, ch, crow, 0)
            pltpu.sync_copy(eb, out.at[pl.ds(b, ch)])
            return carry
        lax.fori_loop(0, nchunk, chunk, 0)

    return pl.kernel(body, out_type=out_type, mesh=mesh,
                     scratch_types=scratch)


def kernel(x, edge_attr, edge_index, Wm, bm, Wa, ba, We, be):
    n, d = x.shape
    e, de = edge_attr.shape
    nl = Wm.shape[0]
    assert de == LANES

    src = edge_index[0].astype(jnp.int32)
    dst = edge_index[1].astype(jnp.int32)

    bn = 400
    be_blk = 2000

    msg0 = _make_msg_kernel(n, d, e, with_cnt=True)
    msg = _make_msg_kernel(n, d, e, with_cnt=False)
    edge_upd = _make_edge_update_kernel(n, de, e)

    h = x
    ea = edge_attr
    hx = _tc_node_matmul(x, Wm[0][:d], bn)
    cp = None
    for l in range(nl):
        last = l == nl - 1
        if not last:
            eaw, eaw2 = _edge_prep(ea, Wm[l][d:], bm[l][None],
                                   We[l][2 * d:], be[l][None], be_blk)
        else:
            eaw = _edge_prep(ea, Wm[l][d:], bm[l][None], None, None, be_blk)
        if l == 0:
            sp, cp = msg0(hx, eaw, src, dst)
        else:
            sp = msg(hx, eaw, src, dst)
        if not last:
            h, hx, hi, hj = _update(sp, cp, h, Wa[l][:d], Wa[l][d:],
                                    ba[l][None], Wm[l + 1][:d],
                                    We[l][:d], We[l][d:2 * d], bn)
            ea = edge_upd(hi, hj, eaw2, src, dst)
        else:
            h = _update(sp, cp, h, Wa[l][:d], Wa[l][d:], ba[l][None],
                        None, None, None, bn)
    return h


# speedup vs baseline: 1.9236x; 1.9236x over previous
import functools

import jax
import jax.numpy as jnp
from jax import lax
from jax.experimental import pallas as pl
from jax.experimental.pallas import tpu as pltpu
from jax.experimental.pallas import tpu_sc as plsc

NC = 2
NS = 16
LANES = 16


def _prep0_body(x_ref, w_ref, o_ref):
    o_ref[...] = jnp.dot(x_ref[...], w_ref[...],
                         preferred_element_type=jnp.float32)


def _tc_node_matmul(x, w, bn):
    n, d = x.shape
    return pl.pallas_call(
        _prep0_body,
        grid=(n // bn,),
        in_specs=[
            pl.BlockSpec((bn, d), lambda i: (i, 0)),
            pl.BlockSpec((d, w.shape[1]), lambda i: (0, 0)),
        ],
        out_specs=pl.BlockSpec((bn, w.shape[1]), lambda i: (i, 0)),
        out_shape=jax.ShapeDtypeStruct((n, w.shape[1]), jnp.float32),
    )(x, w)


def _edge_prep2_body(de, ea_ref, wme_ref, bm_ref, wee_ref, be_ref,
                     eaw_ref, eaw2_ref):
    ea = ea_ref[...][:, 0:de]
    eaw_ref[...] = jnp.dot(ea, wme_ref[...],
                           preferred_element_type=jnp.float32) + bm_ref[...]
    eaw2_ref[...] = jnp.dot(ea, wee_ref[...],
                            preferred_element_type=jnp.float32) + be_ref[...]


def _edge_prep1_body(de, ea_ref, wme_ref, bm_ref, eaw_ref):
    ea = ea_ref[...][:, 0:de]
    eaw_ref[...] = jnp.dot(ea, wme_ref[...],
                           preferred_element_type=jnp.float32) + bm_ref[...]


def _edge_prep(ea, de, wme, bmv, wee_p, bev_p, be_blk):
    e, din = ea.shape
    d = wme.shape[1]
    grid = (e // be_blk,)
    if wee_p is None:
        return pl.pallas_call(
            functools.partial(_edge_prep1_body, de),
            grid=grid,
            in_specs=[
                pl.BlockSpec((be_blk, din), lambda i: (i, 0)),
                pl.BlockSpec((de, d), lambda i: (0, 0)),
                pl.BlockSpec((1, d), lambda i: (0, 0)),
            ],
            out_specs=pl.BlockSpec((be_blk, d), lambda i: (i, 0)),
            out_shape=jax.ShapeDtypeStruct((e, d), jnp.float32),
        )(ea, wme, bmv)
    dp = wee_p.shape[1]
    return pl.pallas_call(
        functools.partial(_edge_prep2_body, de),
        grid=grid,
        in_specs=[
            pl.BlockSpec((be_blk, din), lambda i: (i, 0)),
            pl.BlockSpec((de, d), lambda i: (0, 0)),
            pl.BlockSpec((1, d), lambda i: (0, 0)),
            pl.BlockSpec((de, dp), lambda i: (0, 0)),
            pl.BlockSpec((1, dp), lambda i: (0, 0)),
        ],
        out_specs=[
            pl.BlockSpec((be_blk, d), lambda i: (i, 0)),
            pl.BlockSpec((be_blk, dp), lambda i: (i, 0)),
        ],
        out_shape=[
            jax.ShapeDtypeStruct((e, d), jnp.float32),
            jax.ShapeDtypeStruct((e, dp), jnp.float32),
        ],
    )(ea, wme, bmv, wee_p, bev_p)


def _update2_body(sp_ref, cp_ref, h_ref, waa_ref, wah_ref, ba_ref,
                  wmxn_ref, wij_ref,
                  hn_ref, hxn_ref, hij_ref):
    s = sp_ref[0] + sp_ref[1]
    cnt = cp_ref[0, :, 0:1] + cp_ref[1, :, 0:1]
    agg = s * (1.0 / jnp.maximum(cnt, 1.0))
    u = jnp.dot(agg, waa_ref[...], preferred_element_type=jnp.float32)
    u = u + jnp.dot(h_ref[...], wah_ref[...],
                    preferred_element_type=jnp.float32)
    u = jnp.maximum(u + ba_ref[...], 0.0)
    nn = jnp.sqrt(jnp.sum(u * u, axis=1, keepdims=True))
    hv = u / jnp.maximum(nn, 1e-12)
    hn_ref[...] = hv
    hxn_ref[...] = jnp.dot(hv, wmxn_ref[...],
                           preferred_element_type=jnp.float32)
    hij_ref[...] = jnp.dot(hv, wij_ref[...],
                           preferred_element_type=jnp.float32)


def _update1_body(sp_ref, cp_ref, h_ref, waa_ref, wah_ref, ba_ref, hn_ref):
    s = sp_ref[0] + sp_ref[1]
    cnt = cp_ref[0, :, 0:1] + cp_ref[1, :, 0:1]
    agg = s * (1.0 / jnp.maximum(cnt, 1.0))
    u = jnp.dot(agg, waa_ref[...], preferred_element_type=jnp.float32)
    u = u + jnp.dot(h_ref[...], wah_ref[...],
                    preferred_element_type=jnp.float32)
    u = jnp.maximum(u + ba_ref[...], 0.0)
    nn = jnp.sqrt(jnp.sum(u * u, axis=1, keepdims=True))
    hn_ref[...] = u / jnp.maximum(nn, 1e-12)


def _update(sp, cp, h, waa, wah, bav, wmxn, wij_p, bn):
    n, d = h.shape
    de = cp.shape[2]
    grid = (n // bn,)
    common_in = [
        pl.BlockSpec((NC, bn, d), lambda i: (0, i, 0)),
        pl.BlockSpec((NC, bn, de), lambda i: (0, i, 0)),
        pl.BlockSpec((bn, d), lambda i: (i, 0)),
        pl.BlockSpec((d, d), lambda i: (0, 0)),
        pl.BlockSpec((d, d), lambda i: (0, 0)),
        pl.BlockSpec((1, d), lambda i: (0, 0)),
    ]
    if wmxn is None:
        return pl.pallas_call(
            _update1_body,
            grid=grid,
            in_specs=common_in,
            out_specs=pl.BlockSpec((bn, d), lambda i: (i, 0)),
            out_shape=jax.ShapeDtypeStruct((n, d), jnp.float32),
        )(sp, cp, h, waa, wah, bav)
    dp = wij_p.shape[1]
    return pl.pallas_call(
        _update2_body,
        grid=grid,
        in_specs=common_in + [
            pl.BlockSpec((d, d), lambda i: (0, 0)),
            pl.BlockSpec((d, dp), lambda i: (0, 0)),
        ],
        out_specs=[
            pl.BlockSpec((bn, d), lambda i: (i, 0)),
            pl.BlockSpec((bn, d), lambda i: (i, 0)),
            pl.BlockSpec((bn, dp), lambda i: (i, 0)),
        ],
        out_shape=[
            jax.ShapeDtypeStruct((n, d), jnp.float32),
            jax.ShapeDtypeStruct((n, d), jnp.float32),
            jax.ShapeDtypeStruct((n, dp), jnp.float32),
        ],
    )(sp, cp, h, waa, wah, bav, wmxn, wij_p)


def _padded_rows(nn):
    rpt = -(-nn // NS)
    rpt = -(-rpt // 128) * 128
    return rpt, rpt * NS


def _make_msg_kernel(nn, dd, ee):
    w = NC * NS
    ept = ee // w
    ch = 80
    nchunk = ept // ch
    rpt, nnp = _padded_rows(nn)
    nz = rpt // ch
    assert ept % ch == 0 and rpt % ch == 0 and dd % LANES == 0

    mesh = plsc.VectorSubcoreMesh(core_axis_name="c", subcore_axis_name="s",
                                  num_cores=NC, num_subcores=NS)

    out_type = jax.ShapeDtypeStruct((NC, nnp, dd), jnp.float32)
    scratch = [
        pltpu.VMEM((ch,), jnp.int32),
        pltpu.VMEM((1, ch), jnp.int32),
        pltpu.VMEM((ch, dd), jnp.float32),
        pltpu.VMEM((ch, dd), jnp.float32),
        pltpu.VMEM_SHARED((nnp, dd), jnp.float32),
        pltpu.SemaphoreType.DMA,
    ]

    def body(hx, eaw, srcr, dstr, out_s, srci, dsti, rows, eawb, acc, sem):
        c = lax.axis_index("c")
        s = lax.axis_index("s")
        ncol = dd // LANES

        def zrow(r, carry):
            for cc in range(ncol):
                rows[r, cc * LANES:(cc + 1) * LANES] = jnp.zeros(
                    (LANES,), jnp.float32)
            return carry
        lax.fori_loop(0, ch, zrow, 0)

        base_row = s * rpt
        for z in range(nz):
            pltpu.sync_copy(rows, acc.at[pl.ds(base_row + z * ch, ch)])

        plsc.subcore_barrier()

        ebase = (c * NS + s) * ept

        def chunk(j, carry):
            b = ebase + j * ch
            pltpu.sync_copy(srcr.at[pl.ds(b, ch)], srci)
            pltpu.sync_copy(dstr.at[pl.ds(b, ch)], dsti.at[0])
            pltpu.async_copy(hx.at[srci], rows, sem).wait()
            pltpu.sync_copy(eaw.at[pl.ds(b, ch)], eawb)

            def crow(r, carry2):
                for cc in range(ncol):
                    sl = slice(cc * LANES, (cc + 1) * LANES)
                    rows[r, sl] = jnp.maximum(rows[r, sl] + eawb[r, sl], 0.0)
                return carry2
            lax.fori_loop(0, ch, crow, 0)

            pltpu.sync_copy(rows, acc.at[dsti.at[0]], add=True)
            return carry
        lax.fori_loop(0, nchunk, chunk, 0)

        plsc.subcore_barrier()
        pltpu.sync_copy(acc.at[pl.ds(base_row, rpt)],
                        out_s.at[c, pl.ds(base_row, rpt)])

    return pl.kernel(body, out_type=out_type, mesh=mesh,
                     scratch_types=scratch)


def _make_cnt_kernel(nn, ee, dd):
    w = NC * NS
    ept = ee // w
    ch = 80
    nchunk = ept // ch
    rpt, nnp = _padded_rows(nn)
    nz = rpt // ch
    assert ept % ch == 0 and rpt % ch == 0

    mesh = plsc.VectorSubcoreMesh(core_axis_name="c", subcore_axis_name="s",
                                  num_cores=NC, num_subcores=NS)
    out_type = jax.ShapeDtypeStruct((NC, nnp, dd), jnp.float32)
    scratch = [
        pltpu.VMEM((1, ch), jnp.int32),
        pltpu.VMEM((ch, dd), jnp.float32),
        pltpu.VMEM((ch, dd), jnp.float32),
        pltpu.VMEM_SHARED((nnp, dd), jnp.float32),
    ]

    def body(dstr, out_c, dsti, ones, zbuf, acccnt):
        c = lax.axis_index("c")
        s = lax.axis_index("s")

        def fill(r, carry):
            for cc in range(dd // LANES):
                sl = slice(cc * LANES, (cc + 1) * LANES)
                ones[r, sl] = jnp.ones((LANES,), jnp.float32)
                zbuf[r, sl] = jnp.zeros((LANES,), jnp.float32)
            return carry
        lax.fori_loop(0, ch, fill, 0)

        base_row = s * rpt
        for z in range(nz):
            pltpu.sync_copy(zbuf, acccnt.at[pl.ds(base_row + z * ch, ch)])

        plsc.subcore_barrier()

        ebase = (c * NS + s) * ept

        def chunk(j, carry):
            b = ebase + j * ch
            pltpu.sync_copy(dstr.at[pl.ds(b, ch)], dsti.at[0])
            pltpu.sync_copy(ones, acccnt.at[dsti.at[0]], add=True)
            return carry
        lax.fori_loop(0, nchunk, chunk, 0)

        plsc.subcore_barrier()
        pltpu.sync_copy(acccnt.at[pl.ds(base_row, rpt)],
                        out_c.at[c, pl.ds(base_row, rpt)])

    return pl.kernel(body, out_type=out_type, mesh=mesh,
                     scratch_types=scratch)


def _make_edge_update_kernel(nn, dp, ee):
    w = NC * NS
    ept = ee // w
    ch = 80
    nchunk = ept // ch
    assert ept % ch == 0

    mesh = plsc.VectorSubcoreMesh(core_axis_name="c", subcore_axis_name="s",
                                  num_cores=NC, num_subcores=NS)
    out_type = jax.ShapeDtypeStruct((ee, dp), jnp.float32)
    scratch = [
        pltpu.VMEM((ch,), jnp.int32),
        pltpu.VMEM((ch,), jnp.int32),
        pltpu.VMEM((ch, dp), jnp.float32),
        pltpu.VMEM((ch, dp), jnp.float32),
        pltpu.VMEM((ch, dp), jnp.float32),
        pltpu.SemaphoreType.DMA,
        pltpu.SemaphoreType.DMA,
    ]

    def body(hij, eaw2, srcr, dstr, out, srci, dsti, g1, g2, eb,
             sem1, sem2):
        c = lax.axis_index("c")
        s = lax.axis_index("s")
        ebase = (c * NS + s) * ept

        def chunk(j, carry):
            b = ebase + j * ch
            pltpu.sync_copy(srcr.at[pl.ds(b, ch)], srci)
            pltpu.sync_copy(dstr.at[pl.ds(b, ch)], dsti)
            cp1 = pltpu.async_copy(hij.at[srci], g1, sem1)
            cp2 = pltpu.async_copy(hij.at[dsti], g2, sem2)
            pltpu.sync_copy(eaw2.at[pl.ds(b, ch)], eb)
            cp1.wait()
            cp2.wait()

            def crow(r, carry2):
                v = (eb[r, 0:LANES] + g1[r, 0:LANES]
                     + g2[r, LANES:2 * LANES])
                eb[r, 0:LANES] = jnp.maximum(v, 0.0)
                return carry2
            lax.fori_loop(0, ch, crow, 0)
            pltpu.sync_copy(eb, out.at[pl.ds(b, ch)])
            return carry
        lax.fori_loop(0, nchunk, chunk, 0)

    return pl.kernel(body, out_type=out_type, mesh=mesh,
                     scratch_types=scratch)


def kernel(x, edge_attr, edge_index, Wm, bm, Wa, ba, We, be):
    n, d = x.shape
    e, de = edge_attr.shape
    nl = Wm.shape[0]
    assert de == LANES

    src = edge_index[0].astype(jnp.int32)
    dst = edge_index[1].astype(jnp.int32)

    bn = 400
    be_blk = 2000

    dp = 128

    msg = _make_msg_kernel(n, d, e)
    cntk = _make_cnt_kernel(n, e, d)
    edge_upd = _make_edge_update_kernel(n, dp, e)

    h = x
    ea = edge_attr
    hx = _tc_node_matmul(x, Wm[0][:d], bn)
    cp = cntk(dst)
    for l in range(nl):
        last = l == nl - 1
        if not last:
            wee_p = jnp.pad(We[l][2 * d:], ((0, 0), (0, dp - de)))
            bev_p = jnp.pad(be[l], (0, dp - de))[None]
            eaw, eaw2 = _edge_prep(ea, de, Wm[l][d:], bm[l][None],
                                   wee_p, bev_p, be_blk)
        else:
            eaw = _edge_prep(ea, de, Wm[l][d:], bm[l][None],
                             None, None, be_blk)
        sp = msg(hx, eaw, src, dst)
        if not last:
            wij_p = jnp.pad(
                jnp.concatenate([We[l][:d], We[l][d:2 * d]], axis=1),
                ((0, 0), (0, dp - 2 * de)))
            h, hx, hij = _update(sp, cp, h, Wa[l][:d], Wa[l][d:],
                                 ba[l][None], Wm[l + 1][:d], wij_p, bn)
            ea = edge_upd(hij, eaw2, src, dst)
        else:
            h = _update(sp, cp, h, Wa[l][:d], Wa[l][d:], ba[l][None],
                        None, None, bn)
    return h


# speedup vs baseline: 2.4158x; 1.2559x over previous
import functools

import jax
import jax.numpy as jnp
from jax import lax
from jax.experimental import pallas as pl
from jax.experimental.pallas import tpu as pltpu
from jax.experimental.pallas import tpu_sc as plsc

NC = 2
NS = 16
LANES = 16


def _prep0_body(x_ref, w_ref, o_ref):
    o_ref[...] = jnp.dot(x_ref[...], w_ref[...],
                         preferred_element_type=jnp.float32)


def _tc_node_matmul(x, w, bn):
    n, d = x.shape
    return pl.pallas_call(
        _prep0_body,
        grid=(n // bn,),
        in_specs=[
            pl.BlockSpec((bn, d), lambda i: (i, 0)),
            pl.BlockSpec((d, w.shape[1]), lambda i: (0, 0)),
        ],
        out_specs=pl.BlockSpec((bn, w.shape[1]), lambda i: (i, 0)),
        out_shape=jax.ShapeDtypeStruct((n, w.shape[1]), jnp.float32),
    )(x, w)


def _edge_prep2_body(de, ea_ref, wme_ref, bm_ref, wee_ref, be_ref,
                     eaw_ref, eaw2_ref):
    ea = ea_ref[...][:, 0:de]
    eaw_ref[...] = jnp.dot(ea, wme_ref[...],
                           preferred_element_type=jnp.float32) + bm_ref[...]
    eaw2_ref[...] = jnp.dot(ea, wee_ref[...],
                            preferred_element_type=jnp.float32) + be_ref[...]


def _edge_prep1_body(de, ea_ref, wme_ref, bm_ref, eaw_ref):
    ea = ea_ref[...][:, 0:de]
    eaw_ref[...] = jnp.dot(ea, wme_ref[...],
                           preferred_element_type=jnp.float32) + bm_ref[...]


def _edge_prep(ea, de, wme, bmv, wee_p, bev_p, be_blk):
    e, din = ea.shape
    d = wme.shape[1]
    grid = (e // be_blk,)
    if wee_p is None:
        return pl.pallas_call(
            functools.partial(_edge_prep1_body, de),
            grid=grid,
            in_specs=[
                pl.BlockSpec((be_blk, din), lambda i: (i, 0)),
                pl.BlockSpec((de, d), lambda i: (0, 0)),
                pl.BlockSpec((1, d), lambda i: (0, 0)),
            ],
            out_specs=pl.BlockSpec((be_blk, d), lambda i: (i, 0)),
            out_shape=jax.ShapeDtypeStruct((e, d), jnp.float32),
        )(ea, wme, bmv)
    dp = wee_p.shape[1]
    return pl.pallas_call(
        functools.partial(_edge_prep2_body, de),
        grid=grid,
        in_specs=[
            pl.BlockSpec((be_blk, din), lambda i: (i, 0)),
            pl.BlockSpec((de, d), lambda i: (0, 0)),
            pl.BlockSpec((1, d), lambda i: (0, 0)),
            pl.BlockSpec((de, dp), lambda i: (0, 0)),
            pl.BlockSpec((1, dp), lambda i: (0, 0)),
        ],
        out_specs=[
            pl.BlockSpec((be_blk, d), lambda i: (i, 0)),
            pl.BlockSpec((be_blk, dp), lambda i: (i, 0)),
        ],
        out_shape=[
            jax.ShapeDtypeStruct((e, d), jnp.float32),
            jax.ShapeDtypeStruct((e, dp), jnp.float32),
        ],
    )(ea, wme, bmv, wee_p, bev_p)


def _update2_body(sp_ref, cp_ref, h_ref, waa_ref, wah_ref, ba_ref,
                  wmxn_ref, wij_ref,
                  hn_ref, hxn_ref, hij_ref):
    s = sp_ref[0] + sp_ref[1]
    cnt = cp_ref[0, :, 0:1] + cp_ref[1, :, 0:1]
    agg = s * (1.0 / jnp.maximum(cnt, 1.0))
    u = jnp.dot(agg, waa_ref[...], preferred_element_type=jnp.float32)
    u = u + jnp.dot(h_ref[...], wah_ref[...],
                    preferred_element_type=jnp.float32)
    u = jnp.maximum(u + ba_ref[...], 0.0)
    nn = jnp.sqrt(jnp.sum(u * u, axis=1, keepdims=True))
    hv = u / jnp.maximum(nn, 1e-12)
    hn_ref[...] = hv
    hxn_ref[...] = jnp.dot(hv, wmxn_ref[...],
                           preferred_element_type=jnp.float32)
    hij_ref[...] = jnp.dot(hv, wij_ref[...],
                           preferred_element_type=jnp.float32)


def _update1_body(sp_ref, cp_ref, h_ref, waa_ref, wah_ref, ba_ref, hn_ref):
    s = sp_ref[0] + sp_ref[1]
    cnt = cp_ref[0, :, 0:1] + cp_ref[1, :, 0:1]
    agg = s * (1.0 / jnp.maximum(cnt, 1.0))
    u = jnp.dot(agg, waa_ref[...], preferred_element_type=jnp.float32)
    u = u + jnp.dot(h_ref[...], wah_ref[...],
                    preferred_element_type=jnp.float32)
    u = jnp.maximum(u + ba_ref[...], 0.0)
    nn = jnp.sqrt(jnp.sum(u * u, axis=1, keepdims=True))
    hn_ref[...] = u / jnp.maximum(nn, 1e-12)


def _update(sp, cp, h, waa, wah, bav, wmxn, wij_p, bn):
    n, d = h.shape
    de = cp.shape[2]
    grid = (n // bn,)
    common_in = [
        pl.BlockSpec((NC, bn, d), lambda i: (0, i, 0)),
        pl.BlockSpec((NC, bn, de), lambda i: (0, i, 0)),
        pl.BlockSpec((bn, d), lambda i: (i, 0)),
        pl.BlockSpec((d, d), lambda i: (0, 0)),
        pl.BlockSpec((d, d), lambda i: (0, 0)),
        pl.BlockSpec((1, d), lambda i: (0, 0)),
    ]
    if wmxn is None:
        return pl.pallas_call(
            _update1_body,
            grid=grid,
            in_specs=common_in,
            out_specs=pl.BlockSpec((bn, d), lambda i: (i, 0)),
            out_shape=jax.ShapeDtypeStruct((n, d), jnp.float32),
        )(sp, cp, h, waa, wah, bav)
    dp = wij_p.shape[1]
    return pl.pallas_call(
        _update2_body,
        grid=grid,
        in_specs=common_in + [
            pl.BlockSpec((d, d), lambda i: (0, 0)),
            pl.BlockSpec((d, dp), lambda i: (0, 0)),
        ],
        out_specs=[
            pl.BlockSpec((bn, d), lambda i: (i, 0)),
            pl.BlockSpec((bn, d), lambda i: (i, 0)),
            pl.BlockSpec((bn, dp), lambda i: (i, 0)),
        ],
        out_shape=[
            jax.ShapeDtypeStruct((n, d), jnp.float32),
            jax.ShapeDtypeStruct((n, d), jnp.float32),
            jax.ShapeDtypeStruct((n, dp), jnp.float32),
        ],
    )(sp, cp, h, waa, wah, bav, wmxn, wij_p)


def _padded_rows(nn):
    rpt = -(-nn // NS)
    rpt = -(-rpt // 128) * 128
    return rpt, rpt * NS


def _make_msg_kernel(nn, dd, ee):
    w = NC * NS
    ept = ee // w
    ch = 40
    nchunk = ept // ch
    rpt, nnp = _padded_rows(nn)
    nz = rpt // ch
    assert ept % ch == 0 and rpt % ch == 0 and dd % LANES == 0
    assert nchunk % 2 == 0 and nchunk >= 4

    mesh = plsc.VectorSubcoreMesh(core_axis_name="c", subcore_axis_name="s",
                                  num_cores=NC, num_subcores=NS)

    out_type = jax.ShapeDtypeStruct((NC, nnp, dd), jnp.float32)
    scratch = [
        pltpu.VMEM((2, 2, ch), jnp.int32),
        pltpu.VMEM((2, ch, dd), jnp.float32),
        pltpu.VMEM((2, ch, dd), jnp.float32),
        pltpu.VMEM_SHARED((nnp, dd), jnp.float32),
        pltpu.SemaphoreType.DMA,
        pltpu.SemaphoreType.DMA,
        pltpu.SemaphoreType.DMA,
        pltpu.SemaphoreType.DMA,
        pltpu.SemaphoreType.DMA,
        pltpu.SemaphoreType.DMA,
    ]

    def body(hx, eaw, srcr, dstr, out_s, ibuf, rows, eawb, acc,
             sg0, sg1, se0, se1, ss0, ss1):
        c = lax.axis_index("c")
        s = lax.axis_index("s")
        ncol = dd // LANES
        semg = [sg0, sg1]
        seme = [se0, se1]
        sems = [ss0, ss1]

        def zrow(r, carry):
            for cc in range(ncol):
                rows[0, r, cc * LANES:(cc + 1) * LANES] = jnp.zeros(
                    (LANES,), jnp.float32)
            return carry
        lax.fori_loop(0, ch, zrow, 0)

        base_row = s * rpt
        for z in range(nz):
            pltpu.sync_copy(rows.at[0], acc.at[pl.ds(base_row + z * ch, ch)])

        plsc.subcore_barrier()

        ebase = (c * NS + s) * ept

        def start_chunk(j, b):
            boff = ebase + j * ch
            pltpu.sync_copy(srcr.at[pl.ds(boff, ch)], ibuf.at[b, 0])
            pltpu.sync_copy(dstr.at[pl.ds(boff, ch)], ibuf.at[b, 1])
            pltpu.async_copy(hx.at[ibuf.at[b, 0]], rows.at[b], semg[b])
            pltpu.async_copy(eaw.at[pl.ds(boff, ch)], eawb.at[b], seme[b])

        def process(j, b, prefetch):
            boff = ebase + j * ch
            pltpu.make_async_copy(hx.at[ibuf.at[b, 0]], rows.at[b],
                                  semg[b]).wait()
            pltpu.make_async_copy(eaw.at[pl.ds(boff, ch)], eawb.at[b],
                                  seme[b]).wait()

            def crow(r, carry2):
                for cc in range(ncol):
                    sl = slice(cc * LANES, (cc + 1) * LANES)
                    rows[b, r, sl] = jnp.maximum(
                        rows[b, r, sl] + eawb[b, r, sl], 0.0)
                return carry2
            lax.fori_loop(0, ch, crow, 0)

            scat = pltpu.async_copy(rows.at[b], acc.at[ibuf.at[b, 1]],
                                    sems[b], add=True)
            boff2 = ebase + (j + 2) * ch
            if prefetch:
                pltpu.async_copy(eaw.at[pl.ds(boff2, ch)], eawb.at[b],
                                 seme[b])
            scat.wait()
            if prefetch:
                pltpu.sync_copy(srcr.at[pl.ds(boff2, ch)], ibuf.at[b, 0])
                pltpu.sync_copy(dstr.at[pl.ds(boff2, ch)], ibuf.at[b, 1])
                pltpu.async_copy(hx.at[ibuf.at[b, 0]], rows.at[b], semg[b])

        start_chunk(0, 0)
        start_chunk(1, 1)

        def step(g, carry):
            process(2 * g, 0, True)
            process(2 * g + 1, 1, True)
            return carry
        lax.fori_loop(0, nchunk // 2 - 1, step, 0)
        process(nchunk - 2, 0, False)
        process(nchunk - 1, 1, False)

        plsc.subcore_barrier()
        pltpu.sync_copy(acc.at[pl.ds(base_row, rpt)],
                        out_s.at[c, pl.ds(base_row, rpt)])

    return pl.kernel(body, out_type=out_type, mesh=mesh,
                     scratch_types=scratch)


def _make_cnt_kernel(nn, ee, dd):
    w = NC * NS
    ept = ee // w
    ch = 40
    nchunk = ept // ch
    rpt, nnp = _padded_rows(nn)
    nz = rpt // ch
    assert ept % ch == 0 and rpt % ch == 0
    assert nchunk % 2 == 0 and nchunk >= 4

    mesh = plsc.VectorSubcoreMesh(core_axis_name="c", subcore_axis_name="s",
                                  num_cores=NC, num_subcores=NS)
    out_type = jax.ShapeDtypeStruct((NC, nnp, dd), jnp.float32)
    scratch = [
        pltpu.VMEM((2, 1, ch), jnp.int32),
        pltpu.VMEM((ch, dd), jnp.float32),
        pltpu.VMEM((ch, dd), jnp.float32),
        pltpu.VMEM_SHARED((nnp, dd), jnp.float32),
        pltpu.SemaphoreType.DMA,
        pltpu.SemaphoreType.DMA,
    ]

    def body(dstr, out_c, dsti, ones, zbuf, acccnt, ss0, ss1):
        c = lax.axis_index("c")
        s = lax.axis_index("s")
        sems = [ss0, ss1]

        def fill(r, carry):
            for cc in range(dd // LANES):
                sl = slice(cc * LANES, (cc + 1) * LANES)
                ones[r, sl] = jnp.ones((LANES,), jnp.float32)
                zbuf[r, sl] = jnp.zeros((LANES,), jnp.float32)
            return carry
        lax.fori_loop(0, ch, fill, 0)

        base_row = s * rpt
        for z in range(nz):
            pltpu.sync_copy(zbuf, acccnt.at[pl.ds(base_row + z * ch, ch)])

        plsc.subcore_barrier()

        ebase = (c * NS + s) * ept

        def start_chunk(j, b):
            boff = ebase + j * ch
            pltpu.sync_copy(dstr.at[pl.ds(boff, ch)], dsti.at[b, 0])
            pltpu.async_copy(ones, acccnt.at[dsti.at[b, 0]], sems[b],
                             add=True)

        def process(j, b, prefetch):
            pltpu.make_async_copy(ones, acccnt.at[dsti.at[b, 0]],
                                  sems[b]).wait()
            if prefetch:
                start_chunk(j + 2, b)

        start_chunk(0, 0)
        start_chunk(1, 1)

        def step(g, carry):
            process(2 * g, 0, True)
            process(2 * g + 1, 1, True)
            return carry
        lax.fori_loop(0, nchunk // 2 - 1, step, 0)
        process(nchunk - 2, 0, False)
        process(nchunk - 1, 1, False)

        plsc.subcore_barrier()
        pltpu.sync_copy(acccnt.at[pl.ds(base_row, rpt)],
                        out_c.at[c, pl.ds(base_row, rpt)])

    return pl.kernel(body, out_type=out_type, mesh=mesh,
                     scratch_types=scratch)


def _make_edge_update_kernel(nn, dp, ee):
    w = NC * NS
    ept = ee // w
    ch = 40
    nchunk = ept // ch
    assert ept % ch == 0
    assert nchunk % 2 == 0 and nchunk >= 4

    mesh = plsc.VectorSubcoreMesh(core_axis_name="c", subcore_axis_name="s",
                                  num_cores=NC, num_subcores=NS)
    out_type = jax.ShapeDtypeStruct((ee, dp), jnp.float32)
    scratch = [
        pltpu.VMEM((2, 2, ch), jnp.int32),
        pltpu.VMEM((2, ch, dp), jnp.float32),
        pltpu.VMEM((2, ch, dp), jnp.float32),
        pltpu.VMEM((2, ch, dp), jnp.float32),
        pltpu.SemaphoreType.DMA,
        pltpu.SemaphoreType.DMA,
        pltpu.SemaphoreType.DMA,
        pltpu.SemaphoreType.DMA,
        pltpu.SemaphoreType.DMA,
        pltpu.SemaphoreType.DMA,
        pltpu.SemaphoreType.DMA,
        pltpu.SemaphoreType.DMA,
    ]

    def body(hij, eaw2, srcr, dstr, out, ibuf, g1, g2, eb,
             s10, s11, s20, s21, se0, se1, so0, so1):
        c = lax.axis_index("c")
        s = lax.axis_index("s")
        sem1 = [s10, s11]
        sem2 = [s20, s21]
        seme = [se0, se1]
        semo = [so0, so1]
        ebase = (c * NS + s) * ept

        def start_chunk(j, b):
            boff = ebase + j * ch
            pltpu.sync_copy(srcr.at[pl.ds(boff, ch)], ibuf.at[b, 0])
            pltpu.sync_copy(dstr.at[pl.ds(boff, ch)], ibuf.at[b, 1])
            pltpu.async_copy(hij.at[ibuf.at[b, 0]], g1.at[b], sem1[b])
            pltpu.async_copy(hij.at[ibuf.at[b, 1]], g2.at[b], sem2[b])
            pltpu.async_copy(eaw2.at[pl.ds(boff, ch)], eb.at[b], seme[b])

        def process(j, b, prefetch):
            boff = ebase + j * ch
            pltpu.make_async_copy(hij.at[ibuf.at[b, 0]], g1.at[b],
                                  sem1[b]).wait()
            pltpu.make_async_copy(hij.at[ibuf.at[b, 1]], g2.at[b],
                                  sem2[b]).wait()
            pltpu.make_async_copy(eaw2.at[pl.ds(boff, ch)], eb.at[b],
                                  seme[b]).wait()

            def crow(r, carry2):
                v = (eb[b, r, 0:LANES] + g1[b, r, 0:LANES]
                     + g2[b, r, LANES:2 * LANES])
                eb[b, r, 0:LANES] = jnp.maximum(v, 0.0)
                return carry2
            lax.fori_loop(0, ch, crow, 0)
            sto = pltpu.async_copy(eb.at[b], out.at[pl.ds(boff, ch)],
                                   semo[b])
            boff2 = boff + 2 * ch
            if prefetch:
                pltpu.sync_copy(srcr.at[pl.ds(boff2, ch)], ibuf.at[b, 0])
                pltpu.sync_copy(dstr.at[pl.ds(boff2, ch)], ibuf.at[b, 1])
                pltpu.async_copy(hij.at[ibuf.at[b, 0]], g1.at[b], sem1[b])
                pltpu.async_copy(hij.at[ibuf.at[b, 1]], g2.at[b], sem2[b])
            sto.wait()
            if prefetch:
                pltpu.async_copy(eaw2.at[pl.ds(boff2, ch)], eb.at[b],
                                 seme[b])

        start_chunk(0, 0)
        start_chunk(1, 1)

        def step(g, carry):
            process(2 * g, 0, True)
            process(2 * g + 1, 1, True)
            return carry
        lax.fori_loop(0, nchunk // 2 - 1, step, 0)
        process(nchunk - 2, 0, False)
        process(nchunk - 1, 1, False)

    return pl.kernel(body, out_type=out_type, mesh=mesh,
                     scratch_types=scratch)


def kernel(x, edge_attr, edge_index, Wm, bm, Wa, ba, We, be):
    n, d = x.shape
    e, de = edge_attr.shape
    nl = Wm.shape[0]
    assert de == LANES

    src = edge_index[0].astype(jnp.int32)
    dst = edge_index[1].astype(jnp.int32)

    bn = 400
    be_blk = 2000

    dp = 128

    msg = _make_msg_kernel(n, d, e)
    cntk = _make_cnt_kernel(n, e, d)
    edge_upd = _make_edge_update_kernel(n, dp, e)

    h = x
    ea = edge_attr
    hx = _tc_node_matmul(x, Wm[0][:d], bn)
    cp = cntk(dst)
    for l in range(nl):
        last = l == nl - 1
        if not last:
            wee_p = jnp.pad(We[l][2 * d:], ((0, 0), (0, dp - de)))
            bev_p = jnp.pad(be[l], (0, dp - de))[None]
            eaw, eaw2 = _edge_prep(ea, de, Wm[l][d:], bm[l][None],
                                   wee_p, bev_p, be_blk)
        else:
            eaw = _edge_prep(ea, de, Wm[l][d:], bm[l][None],
                             None, None, be_blk)
        sp = msg(hx, eaw, src, dst)
        if not last:
            wij_p = jnp.pad(
                jnp.concatenate([We[l][:d], We[l][d:2 * d]], axis=1),
                ((0, 0), (0, dp - 2 * de)))
            h, hx, hij = _update(sp, cp, h, Wa[l][:d], Wa[l][d:],
                                 ba[l][None], Wm[l + 1][:d], wij_p, bn)
            ea = edge_upd(hij, eaw2, src, dst)
        else:
            h = _update(sp, cp, h, Wa[l][:d], Wa[l][d:], ba[l][None],
                        None, None, bn)
    return h


# speedup vs baseline: 3.2495x; 1.3451x over previous
import functools

import jax
import jax.numpy as jnp
from jax import lax
from jax.experimental import pallas as pl
from jax.experimental.pallas import tpu as pltpu
from jax.experimental.pallas import tpu_sc as plsc

NC = 2
NS = 16
LANES = 16


def _prep0_body(x_ref, w_ref, o_ref):
    o_ref[...] = jnp.dot(x_ref[...], w_ref[...],
                         preferred_element_type=jnp.float32)


def _tc_node_matmul(x, w, bn):
    n, d = x.shape
    return pl.pallas_call(
        _prep0_body,
        grid=(n // bn,),
        in_specs=[
            pl.BlockSpec((bn, d), lambda i: (i, 0)),
            pl.BlockSpec((d, w.shape[1]), lambda i: (0, 0)),
        ],
        out_specs=pl.BlockSpec((bn, w.shape[1]), lambda i: (i, 0)),
        out_shape=jax.ShapeDtypeStruct((n, w.shape[1]), jnp.float32),
    )(x, w)


def _edge_prep2_body(de, ea_ref, wee_ref, be_ref, eaw2_ref):
    ea = ea_ref[...][:, 0:de]
    eaw2_ref[...] = jnp.dot(ea, wee_ref[...],
                            preferred_element_type=jnp.float32) + be_ref[...]


def _edge_prep1_body(de, ea_ref, wme_ref, bm_ref, eaw_ref):
    ea = ea_ref[...][:, 0:de]
    eaw_ref[...] = jnp.dot(ea, wme_ref[...],
                           preferred_element_type=jnp.float32) + bm_ref[...]


def _edge_prep(ea, de, wme, bmv, be_blk):
    e, din = ea.shape
    d = wme.shape[1]
    return pl.pallas_call(
        functools.partial(_edge_prep1_body, de),
        grid=(e // be_blk,),
        in_specs=[
            pl.BlockSpec((be_blk, din), lambda i: (i, 0)),
            pl.BlockSpec((de, d), lambda i: (0, 0)),
            pl.BlockSpec((1, d), lambda i: (0, 0)),
        ],
        out_specs=pl.BlockSpec((be_blk, d), lambda i: (i, 0)),
        out_shape=jax.ShapeDtypeStruct((e, d), jnp.float32),
    )(ea, wme, bmv)


def _edge_prep2(ea, de, wee_p, bev_p, be_blk):
    e, din = ea.shape
    dp = wee_p.shape[1]
    return pl.pallas_call(
        functools.partial(_edge_prep2_body, de),
        grid=(e // be_blk,),
        in_specs=[
            pl.BlockSpec((be_blk, din), lambda i: (i, 0)),
            pl.BlockSpec((de, dp), lambda i: (0, 0)),
            pl.BlockSpec((1, dp), lambda i: (0, 0)),
        ],
        out_specs=pl.BlockSpec((be_blk, dp), lambda i: (i, 0)),
        out_shape=jax.ShapeDtypeStruct((e, dp), jnp.float32),
    )(ea, wee_p, bev_p)


def _update2_body(sp_ref, cp_ref, h_ref, waa_ref, wah_ref, ba_ref,
                  wmxn_ref, wij_ref,
                  hn_ref, hxn_ref, hij_ref):
    s = sp_ref[0] + sp_ref[1]
    cnt = cp_ref[0, :, 0:1] + cp_ref[1, :, 0:1]
    agg = s * (1.0 / jnp.maximum(cnt, 1.0))
    u = jnp.dot(agg, waa_ref[...], preferred_element_type=jnp.float32)
    u = u + jnp.dot(h_ref[...], wah_ref[...],
                    preferred_element_type=jnp.float32)
    u = jnp.maximum(u + ba_ref[...], 0.0)
    nn = jnp.sqrt(jnp.sum(u * u, axis=1, keepdims=True))
    hv = u / jnp.maximum(nn, 1e-12)
    hn_ref[...] = hv
    hxn_ref[...] = jnp.dot(hv, wmxn_ref[...],
                           preferred_element_type=jnp.float32)
    hij_ref[...] = jnp.dot(hv, wij_ref[...],
                           preferred_element_type=jnp.float32)


def _update1_body(sp_ref, cp_ref, h_ref, waa_ref, wah_ref, ba_ref, hn_ref):
    s = sp_ref[0] + sp_ref[1]
    cnt = cp_ref[0, :, 0:1] + cp_ref[1, :, 0:1]
    agg = s * (1.0 / jnp.maximum(cnt, 1.0))
    u = jnp.dot(agg, waa_ref[...], preferred_element_type=jnp.float32)
    u = u + jnp.dot(h_ref[...], wah_ref[...],
                    preferred_element_type=jnp.float32)
    u = jnp.maximum(u + ba_ref[...], 0.0)
    nn = jnp.sqrt(jnp.sum(u * u, axis=1, keepdims=True))
    hn_ref[...] = u / jnp.maximum(nn, 1e-12)


def _update(sp, cp, h, waa, wah, bav, wmxn, wij_p, bn):
    n, d = h.shape
    de = cp.shape[2]
    grid = (n // bn,)
    common_in = [
        pl.BlockSpec((NC, bn, d), lambda i: (0, i, 0)),
        pl.BlockSpec((NC, bn, de), lambda i: (0, i, 0)),
        pl.BlockSpec((bn, d), lambda i: (i, 0)),
        pl.BlockSpec((d, d), lambda i: (0, 0)),
        pl.BlockSpec((d, d), lambda i: (0, 0)),
        pl.BlockSpec((1, d), lambda i: (0, 0)),
    ]
    if wmxn is None:
        return pl.pallas_call(
            _update1_body,
            grid=grid,
            in_specs=common_in,
            out_specs=pl.BlockSpec((bn, d), lambda i: (i, 0)),
            out_shape=jax.ShapeDtypeStruct((n, d), jnp.float32),
        )(sp, cp, h, waa, wah, bav)
    dp = wij_p.shape[1]
    return pl.pallas_call(
        _update2_body,
        grid=grid,
        in_specs=common_in + [
            pl.BlockSpec((d, d), lambda i: (0, 0)),
            pl.BlockSpec((d, dp), lambda i: (0, 0)),
        ],
        out_specs=[
            pl.BlockSpec((bn, d), lambda i: (i, 0)),
            pl.BlockSpec((bn, d), lambda i: (i, 0)),
            pl.BlockSpec((bn, dp), lambda i: (i, 0)),
        ],
        out_shape=[
            jax.ShapeDtypeStruct((n, d), jnp.float32),
            jax.ShapeDtypeStruct((n, d), jnp.float32),
            jax.ShapeDtypeStruct((n, dp), jnp.float32),
        ],
    )(sp, cp, h, waa, wah, bav, wmxn, wij_p)


def _padded_rows(nn):
    rpt = -(-nn // NS)
    rpt = -(-rpt // 128) * 128
    return rpt, rpt * NS


def _make_msg_kernel(nn, dd, ee):
    w = NC * NS
    ept = ee // w
    ch = 40
    nchunk = ept // ch
    rpt, nnp = _padded_rows(nn)
    nz = rpt // ch
    assert ept % ch == 0 and rpt % ch == 0 and dd % LANES == 0
    assert nchunk % 2 == 0 and nchunk >= 8

    mesh = plsc.VectorSubcoreMesh(core_axis_name="c", subcore_axis_name="s",
                                  num_cores=NC, num_subcores=NS)

    out_type = jax.ShapeDtypeStruct((NC, nnp, dd), jnp.float32)
    scratch = (
        [pltpu.VMEM((8, 2, ch), jnp.int32),
         pltpu.VMEM((4, ch, dd), jnp.float32),
         pltpu.VMEM((2, ch, dd), jnp.float32),
         pltpu.VMEM_SHARED((nnp, dd), jnp.float32)]
        + [pltpu.SemaphoreType.DMA] * 10
    )

    def body(hx, eaw, srcr, dstr, out_s, ibuf, rows, eawb, acc, *sem):
        c = lax.axis_index("c")
        s = lax.axis_index("s")
        ncol = dd // LANES
        semi = list(sem[0:4])
        semg = list(sem[4:6])
        seme = list(sem[6:8])
        sems = list(sem[8:10])

        def zrow(r, carry):
            for cc in range(ncol):
                rows[0, r, cc * LANES:(cc + 1) * LANES] = jnp.zeros(
                    (LANES,), jnp.float32)
            return carry
        lax.fori_loop(0, ch, zrow, 0)

        base_row = s * rpt
        for z in range(nz):
            pltpu.sync_copy(rows.at[0], acc.at[pl.ds(base_row + z * ch, ch)])

        plsc.subcore_barrier()

        ebase = (c * NS + s) * ept

        def issue_idx(j, s8):
            boff = ebase + j * ch
            pltpu.async_copy(srcr.at[pl.ds(boff, ch)], ibuf.at[s8, 0],
                             semi[s8 % 4])
            pltpu.async_copy(dstr.at[pl.ds(boff, ch)], ibuf.at[s8, 1],
                             semi[s8 % 4])

        def wait_idx(j, s8):
            boff = ebase + j * ch
            pltpu.make_async_copy(srcr.at[pl.ds(boff, ch)], ibuf.at[s8, 0],
                                  semi[s8 % 4]).wait()
            pltpu.make_async_copy(dstr.at[pl.ds(boff, ch)], ibuf.at[s8, 1],
                                  semi[s8 % 4]).wait()

        def process(j, s8, pf_idx, pf_g, drain):
            b = s8 % 2
            s4 = s8 % 4
            boff = ebase + j * ch
            pltpu.make_async_copy(hx.at[ibuf.at[s8, 0]], rows.at[s4],
                                  semg[b]).wait()
            pltpu.make_async_copy(eaw.at[pl.ds(boff, ch)], eawb.at[b],
                                  seme[b]).wait()

            def crow(r, carry2):
                for cc in range(ncol):
                    sl = slice(cc * LANES, (cc + 1) * LANES)
                    rows[s4, r, sl] = jnp.maximum(
                        rows[s4, r, sl] + eawb[b, r, sl], 0.0)
                return carry2
            lax.fori_loop(0, ch, crow, 0)

            if drain:
                pltpu.make_async_copy(
                    rows.at[(s4 + 2) % 4], acc.at[ibuf.at[(s8 + 6) % 8, 1]],
                    sems[b]).wait()
            pltpu.async_copy(rows.at[s4], acc.at[ibuf.at[s8, 1]],
                             sems[b], add=True)
            if pf_g:
                pltpu.async_copy(eaw.at[pl.ds(boff + 2 * ch, ch)],
                                 eawb.at[b], seme[b])
            if pf_idx:
                issue_idx(j + 4, (s8 + 4) % 8)
            if pf_g:
                wait_idx(j + 2, (s8 + 2) % 8)
                pltpu.async_copy(hx.at[ibuf.at[(s8 + 2) % 8, 0]],
                                 rows.at[(s4 + 2) % 4], semg[b])

        for j in range(4):
            issue_idx(j, j)
        for j in range(2):
            wait_idx(j, j)
            pltpu.async_copy(hx.at[ibuf.at[j, 0]], rows.at[j], semg[j])
            pltpu.async_copy(eaw.at[pl.ds(ebase + j * ch, ch)],
                             eawb.at[j], seme[j])

        tail_start = ((nchunk - 4) // 8) * 8
        for j in range(8):
            process(j, j, True, True, j >= 2)

        def step(g, carry):
            for b8 in range(8):
                process(8 * g + b8, b8, True, True, True)
            return carry
        lax.fori_loop(1, tail_start // 8, step, 0)
        for j in range(tail_start, nchunk):
            process(j, j % 8, j + 4 < nchunk, j + 2 < nchunk, True)
        for j in (nchunk - 2, nchunk - 1):
            pltpu.make_async_copy(
                rows.at[j % 4], acc.at[ibuf.at[j % 8, 1]],
                sems[j % 2]).wait()

        plsc.subcore_barrier()
        pltpu.sync_copy(acc.at[pl.ds(base_row, rpt)],
                        out_s.at[c, pl.ds(base_row, rpt)])

    return pl.kernel(body, out_type=out_type, mesh=mesh,
                     scratch_types=scratch)


def _make_cnt_kernel(nn, ee, dd):
    w = NC * NS
    ept = ee // w
    ch = 40
    nchunk = ept // ch
    rpt, nnp = _padded_rows(nn)
    nz = rpt // ch
    assert ept % ch == 0 and rpt % ch == 0
    assert nchunk % 2 == 0 and nchunk >= 8

    mesh = plsc.VectorSubcoreMesh(core_axis_name="c", subcore_axis_name="s",
                                  num_cores=NC, num_subcores=NS)
    out_type = jax.ShapeDtypeStruct((NC, nnp, dd), jnp.float32)
    scratch = (
        [pltpu.VMEM((8, 1, ch), jnp.int32),
         pltpu.VMEM((ch, dd), jnp.float32),
         pltpu.VMEM((ch, dd), jnp.float32),
         pltpu.VMEM_SHARED((nnp, dd), jnp.float32)]
        + [pltpu.SemaphoreType.DMA] * 6
    )

    def body(dstr, out_c, dsti, ones, zbuf, acccnt, *sem):
        c = lax.axis_index("c")
        s = lax.axis_index("s")
        semi = list(sem[0:4])
        sems = list(sem[4:6])

        def fill(r, carry):
            for cc in range(dd // LANES):
                sl = slice(cc * LANES, (cc + 1) * LANES)
                ones[r, sl] = jnp.ones((LANES,), jnp.float32)
                zbuf[r, sl] = jnp.zeros((LANES,), jnp.float32)
            return carry
        lax.fori_loop(0, ch, fill, 0)

        base_row = s * rpt
        for z in range(nz):
            pltpu.sync_copy(zbuf, acccnt.at[pl.ds(base_row + z * ch, ch)])

        plsc.subcore_barrier()

        ebase = (c * NS + s) * ept

        def issue_idx(j, s8):
            pltpu.async_copy(dstr.at[pl.ds(ebase + j * ch, ch)],
                             dsti.at[s8, 0], semi[s8 % 4])

        def process(j, s8, pf_idx, drain):
            b = s8 % 2
            pltpu.make_async_copy(dstr.at[pl.ds(ebase + j * ch, ch)],
                                  dsti.at[s8, 0], semi[s8 % 4]).wait()
            if drain:
                pltpu.make_async_copy(ones, acccnt.at[dsti.at[(s8 + 6) % 8, 0]],
                                      sems[b]).wait()
            pltpu.async_copy(ones, acccnt.at[dsti.at[s8, 0]], sems[b],
                             add=True)
            if pf_idx:
                issue_idx(j + 4, (s8 + 4) % 8)

        for j in range(4):
            issue_idx(j, j)

        tail_start = ((nchunk - 4) // 8) * 8
        for j in range(8):
            process(j, j, True, j >= 2)

        def step(g, carry):
            for b8 in range(8):
                process(8 * g + b8, b8, True, True)
            return carry
        lax.fori_loop(1, tail_start // 8, step, 0)
        for j in range(tail_start, nchunk):
            process(j, j % 8, j + 4 < nchunk, True)
        for j in (nchunk - 2, nchunk - 1):
            pltpu.make_async_copy(ones, acccnt.at[dsti.at[j % 8, 0]],
                                  sems[j % 2]).wait()

        plsc.subcore_barrier()
        pltpu.sync_copy(acccnt.at[pl.ds(base_row, rpt)],
                        out_c.at[c, pl.ds(base_row, rpt)])

    return pl.kernel(body, out_type=out_type, mesh=mesh,
                     scratch_types=scratch)


def _make_edge_update_kernel(nn, dp, ee):
    w = NC * NS
    ept = ee // w
    ch = 40
    nchunk = ept // ch
    assert ept % ch == 0
    assert nchunk % 2 == 0 and nchunk >= 8

    mesh = plsc.VectorSubcoreMesh(core_axis_name="c", subcore_axis_name="s",
                                  num_cores=NC, num_subcores=NS)
    out_type = jax.ShapeDtypeStruct((ee, dp), jnp.float32)
    scratch = (
        [pltpu.VMEM((8, 2, ch), jnp.int32),
         pltpu.VMEM((4, ch, dp), jnp.float32),
         pltpu.VMEM((4, ch, dp), jnp.float32),
         pltpu.VMEM((4, ch, dp), jnp.float32)]
        + [pltpu.SemaphoreType.DMA] * 12
    )

    def body(hij, eaw2, srcr, dstr, out, ibuf, g1, g2, eb, *sem):
        c = lax.axis_index("c")
        s = lax.axis_index("s")
        semi = list(sem[0:4])
        sem1 = list(sem[4:6])
        sem2 = list(sem[6:8])
        seme = list(sem[8:10])
        semo = list(sem[10:12])
        ebase = (c * NS + s) * ept

        def issue_idx(j, s8):
            boff = ebase + j * ch
            pltpu.async_copy(srcr.at[pl.ds(boff, ch)], ibuf.at[s8, 0],
                             semi[s8 % 4])
            pltpu.async_copy(dstr.at[pl.ds(boff, ch)], ibuf.at[s8, 1],
                             semi[s8 % 4])

        def wait_idx(j, s8):
            boff = ebase + j * ch
            pltpu.make_async_copy(srcr.at[pl.ds(boff, ch)],
                                  ibuf.at[s8, 0], semi[s8 % 4]).wait()
            pltpu.make_async_copy(dstr.at[pl.ds(boff, ch)],
                                  ibuf.at[s8, 1], semi[s8 % 4]).wait()

        def issue_loads(j, s8):
            boff = ebase + j * ch
            s4 = s8 % 4
            pltpu.async_copy(hij.at[ibuf.at[s8, 0]], g1.at[s4],
                             sem1[s8 % 2])
            pltpu.async_copy(hij.at[ibuf.at[s8, 1]], g2.at[s4],
                             sem2[s8 % 2])
            pltpu.async_copy(eaw2.at[pl.ds(boff, ch)], eb.at[s4],
                             seme[s8 % 2])

        def process(j, s8, pf_idx, pf_g, drain):
            b = s8 % 2
            s4 = s8 % 4
            boff = ebase + j * ch
            pltpu.make_async_copy(hij.at[ibuf.at[s8, 0]], g1.at[s4],
                                  sem1[b]).wait()
            pltpu.make_async_copy(hij.at[ibuf.at[s8, 1]], g2.at[s4],
                                  sem2[b]).wait()
            pltpu.make_async_copy(eaw2.at[pl.ds(boff, ch)], eb.at[s4],
                                  seme[b]).wait()

            def crow(r, carry2):
                v = (eb[s4, r, 0:LANES] + g1[s4, r, 0:LANES]
                     + g2[s4, r, LANES:2 * LANES])
                eb[s4, r, 0:LANES] = jnp.maximum(v, 0.0)
                return carry2
            lax.fori_loop(0, ch, crow, 0)

            if drain:
                pltpu.make_async_copy(
                    eb.at[(s4 + 2) % 4],
                    out.at[pl.ds(boff - 2 * ch, ch)], semo[b]).wait()
            pltpu.async_copy(eb.at[s4], out.at[pl.ds(boff, ch)], semo[b])
            if pf_idx:
                issue_idx(j + 4, (s8 + 4) % 8)
            if pf_g:
                wait_idx(j + 2, (s8 + 2) % 8)
                issue_loads(j + 2, (s8 + 2) % 8)

        for j in range(4):
            issue_idx(j, j)
        for j in range(2):
            wait_idx(j, j)
            issue_loads(j, j)

        tail_start = ((nchunk - 4) // 8) * 8
        for j in range(8):
            process(j, j, True, True, j >= 2)

        def step(g, carry):
            for b8 in range(8):
                process(8 * g + b8, b8, True, True, True)
            return carry
        lax.fori_loop(1, tail_start // 8, step, 0)
        for j in range(tail_start, nchunk):
            process(j, j % 8, j + 4 < nchunk, j + 2 < nchunk, True)
        for j in (nchunk - 2, nchunk - 1):
            pltpu.make_async_copy(
                eb.at[j % 4], out.at[pl.ds(ebase + j * ch, ch)],
                semo[j % 2]).wait()

    return pl.kernel(body, out_type=out_type, mesh=mesh,
                     scratch_types=scratch)


def kernel(x, edge_attr, edge_index, Wm, bm, Wa, ba, We, be):
    n, d = x.shape
    e, de = edge_attr.shape
    nl = Wm.shape[0]
    assert de == LANES

    src = edge_index[0].astype(jnp.int32)
    dst = edge_index[1].astype(jnp.int32)

    bn = 400
    be_blk = 2000

    dp = 128

    msg = _make_msg_kernel(n, d, e)
    cntk = _make_cnt_kernel(n, e, d)
    edge_upd = _make_edge_update_kernel(n, dp, e)

    h = x
    ea = edge_attr
    hx = _tc_node_matmul(x, Wm[0][:d], bn)
    cp = cntk(dst)
    for l in range(nl):
        last = l == nl - 1
        eaw = _edge_prep(ea, de, Wm[l][d:], bm[l][None], be_blk)
        if not last:
            wee_p = jnp.pad(We[l][2 * d:], ((0, 0), (0, dp - de)))
            bev_p = jnp.pad(be[l], (0, dp - de))[None]
            eaw2 = _edge_prep2(ea, de, wee_p, bev_p, be_blk)
        sp = msg(hx, eaw, src, dst)
        if not last:
            wij_p = jnp.pad(
                jnp.concatenate([We[l][:d], We[l][d:2 * d]], axis=1),
                ((0, 0), (0, dp - 2 * de)))
            h, hx, hij = _update(sp, cp, h, Wa[l][:d], Wa[l][d:],
                                 ba[l][None], Wm[l + 1][:d], wij_p, bn)
            ea = edge_upd(hij, eaw2, src, dst)
        else:
            h = _update(sp, cp, h, Wa[l][:d], Wa[l][d:], ba[l][None],
                        None, None, bn)
    return h
